# baseline (speedup 1.0000x reference)
import functools
import math

import jax
import jax.numpy as jnp
from jax import lax
from jax.experimental import pallas as pl
from jax.experimental.pallas import tpu as pltpu
from jax.experimental.pallas import tpu_sc as plsc

D_MODEL = 64
SCALE = math.sqrt(D_MODEL)

NW = 32
BPW = 128
NB_G = 2
NB_T = 4
LEAD = 2


def _make_kernel(Bb, S, V):
    mesh = plsc.VectorSubcoreMesh(core_axis_name="c", subcore_axis_name="s")

    @functools.partial(
        pl.kernel,
        mesh=mesh,
        out_type=jax.ShapeDtypeStruct((S, D_MODEL // 8, NW, 8, 128),
                                      jnp.float32),
        scratch_types=[
            pltpu.VMEM((S // 8, 1, 8, BPW), jnp.int32),
            [pltpu.VMEM((BPW, D_MODEL), jnp.float32) for _ in range(NB_G)],
            [pltpu.VMEM((D_MODEL // 8, 1, 8, 136), jnp.float32)
             for _ in range(NB_T)],
            [pltpu.SemaphoreType.DMA for _ in range(NB_G)],
            [pltpu.SemaphoreType.DMA for _ in range(NB_T)],
        ],
        compiler_params=pltpu.CompilerParams(use_tc_tiling_on_sc=False,
                                             needs_layout_passes=False),
    )
    def emb_kernel(xt_hbm, lut_hbm, out_hbm, idx_v, gbufs, tbufs,
                   gsems, osems):
        wid = lax.axis_index("s") * 2 + lax.axis_index("c")
        pltpu.sync_copy(xt_hbm.at[:, pl.ds(wid, 1)], idx_v)

        iota = lax.iota(jnp.int32, 16)
        dg_v = [(iota + d0) // 8 for d0 in range(0, D_MODEL, 16)]
        di_v = [(iota + d0) % 8 for d0 in range(0, D_MODEL, 16)]
        zero_v = jnp.zeros((16,), jnp.int32)

        def issue_gather(s, g):
            pltpu.async_copy(lut_hbm.at[idx_v.at[s >> 3, 0, s & 7]],
                             gbufs[g], gsems[g])

        for s in range(LEAD):
            issue_gather(s, s % NB_G)

        @pl.loop(0, S, step=NB_T)
        def block(s0):
            for k in range(NB_T):
                s = s0 + k
                g = k % NB_G
                gbuf, tbuf = gbufs[g], tbufs[k]
                pltpu.make_async_copy(
                    lut_hbm.at[idx_v.at[0, 0, 0]], gbuf, gsems[g]).wait()

                @pl.when(s >= NB_T)
                def _drain():
                    pltpu.make_async_copy(
                        tbufs[k].at[:, :, :, pl.ds(0, 128)],
                        out_hbm.at[0, :, pl.ds(0, 1)],
                        osems[k]).wait()

                @plsc.parallel_loop(0, BPW, unroll=8)
                def trow(b):
                    bs = jnp.full((16,), b, jnp.int32)
                    for j in range(D_MODEL // 16):
                        v = gbuf[b, pl.ds(j * 16, 16)] * SCALE
                        plsc.store_scatter(
                            tbuf, [dg_v[j], zero_v, di_v[j], bs], v)

                pltpu.async_copy(
                    tbuf.at[:, :, :, pl.ds(0, 128)],
                    out_hbm.at[s, :, pl.ds(wid, 1)],
                    osems[k])

                @pl.when(s < S - LEAD)
                def _issue():
                    issue_gather(s + LEAD, g)

        for k in range(NB_T):
            pltpu.make_async_copy(
                tbufs[k].at[:, :, :, pl.ds(0, 128)],
                out_hbm.at[0, :, pl.ds(0, 1)], osems[k]).wait()

    return emb_kernel


def kernel(x, lut):
    Bb, S = x.shape
    V, Dm = lut.shape
    x4 = (jnp.swapaxes(x.astype(jnp.int32), 0, 1)
          .reshape(S // 8, 8, NW, BPW)
          .swapaxes(1, 2))
    out5 = _make_kernel(Bb, S, V)(x4, lut)
    y = out5.transpose(2, 4, 0, 1, 3).reshape(Bb, S, Dm)
    return y


# speedup vs baseline: 1.0033x; 1.0033x over previous
import functools
import math

import jax
import jax.numpy as jnp
from jax import lax
from jax.experimental import pallas as pl
from jax.experimental.pallas import tpu as pltpu
from jax.experimental.pallas import tpu_sc as plsc

D_MODEL = 64
SCALE = math.sqrt(D_MODEL)

NW = 32
BPW = 128
NB_G = 2
NB_T = 4
LEAD = 2


def _make_kernel(Bb, S, V):
    mesh = plsc.VectorSubcoreMesh(core_axis_name="c", subcore_axis_name="s")

    @functools.partial(
        pl.kernel,
        mesh=mesh,
        out_type=jax.ShapeDtypeStruct((S, D_MODEL // 8, NW, 8, 128),
                                      jnp.float32),
        scratch_types=[
            pltpu.VMEM((S // 8, 1, 8, BPW), jnp.int32),
            [pltpu.VMEM((BPW, D_MODEL), jnp.float32) for _ in range(NB_G)],
            [pltpu.VMEM((D_MODEL // 8, 1, 8, 136), jnp.float32)
             for _ in range(NB_T)],
            [pltpu.SemaphoreType.DMA for _ in range(NB_G)],
            [pltpu.SemaphoreType.DMA for _ in range(NB_T)],
        ],
        compiler_params=pltpu.CompilerParams(use_tc_tiling_on_sc=False,
                                             needs_layout_passes=False),
    )
    def emb_kernel(xt_hbm, lut_hbm, out_hbm, idx_v, gbufs, tbufs,
                   gsems, osems):
        wid = lax.axis_index("s") * 2 + lax.axis_index("c")
        pltpu.sync_copy(xt_hbm.at[:, pl.ds(wid, 1)], idx_v)

        iota = lax.iota(jnp.int32, 16)
        dg_v = [(iota + d0) // 8 for d0 in range(0, D_MODEL, 16)]
        di_v = [(iota + d0) % 8 for d0 in range(0, D_MODEL, 16)]
        zero_v = jnp.zeros((16,), jnp.int32)

        def issue_gather(s, g):
            pltpu.async_copy(lut_hbm.at[idx_v.at[s >> 3, 0, s & 7]],
                             gbufs[g], gsems[g])

        for s in range(LEAD):
            issue_gather(s, s % NB_G)

        @pl.loop(0, S, step=NB_T)
        def block(s0):
            for k in range(NB_T):
                s = s0 + k
                g = k % NB_G
                gbuf, tbuf = gbufs[g], tbufs[k]
                pltpu.make_async_copy(
                    lut_hbm.at[idx_v.at[0, 0, 0]], gbuf, gsems[g]).wait()

                @pl.when(s >= NB_T)
                def _drain():
                    pltpu.make_async_copy(
                        tbufs[k].at[:, :, :, pl.ds(0, 128)],
                        out_hbm.at[0, :, pl.ds(0, 1)],
                        osems[k]).wait()

                @plsc.parallel_loop(0, BPW, unroll=8)
                def trow(b):
                    bs = jnp.full((16,), b, jnp.int32)
                    for j in range(D_MODEL // 16):
                        v = gbuf[b, pl.ds(j * 16, 16)] * SCALE
                        plsc.store_scatter(
                            tbuf, [dg_v[j], zero_v, di_v[j], bs], v)

                pltpu.async_copy(
                    tbuf.at[:, :, :, pl.ds(0, 128)],
                    out_hbm.at[s, :, pl.ds(wid, 1)],
                    osems[k])

                @pl.when(s < S - LEAD)
                def _issue():
                    issue_gather(s + LEAD, g)

        for k in range(NB_T):
            pltpu.make_async_copy(
                tbufs[k].at[:, :, :, pl.ds(0, 128)],
                out_hbm.at[0, :, pl.ds(0, 1)], osems[k]).wait()

    return emb_kernel


def kernel(x, lut):
    Bb, S = x.shape
    V, Dm = lut.shape
    xm = x.astype(jnp.int32) & jnp.int32(0x7FFFFFFF)
    x4 = (jnp.swapaxes(xm, 0, 1)
          .reshape(S // 8, 8, NW, BPW)
          .swapaxes(1, 2))
    out5 = _make_kernel(Bb, S, V)(x4, lut)
    y = out5.transpose(2, 4, 0, 1, 3).reshape(Bb, S, Dm)
    return y
